# baseline (speedup 1.0000x reference)
import jax
import jax.numpy as jnp
from jax import lax
from jax.experimental import pallas as pl
from jax.experimental.pallas import tpu as pltpu
from jax.experimental.pallas import tpu_sc as plsc

EMBED = 64
NC, NS = 2, 16
NW = NC * NS
SUB = 128
CHUNK = 512
NSUB = CHUNK // SUB


def _gather_body(table_hbm, idx_hbm, out_hbm, idx_v, rows_v, sem):
    wid = lax.axis_index("s") * NC + lax.axis_index("c")
    n = idx_hbm.shape[0] * idx_hbm.shape[1]
    npw = n // NW
    nchunks = npw // CHUNK
    row0 = wid * (npw // SUB)

    @pl.loop(0, nchunks)
    def _chunk(i):
        r = row0 + i * NSUB
        pltpu.sync_copy(idx_hbm.at[pl.ds(r, NSUB)], idx_v)
        copies = [
            pltpu.async_copy(
                table_hbm.at[idx_v.at[j]],
                rows_v.at[pl.ds(j * SUB, SUB)],
                sem,
            )
            for j in range(NSUB)
        ]
        for c in copies:
            c.wait()
        pltpu.sync_copy(rows_v, out_hbm.at[pl.ds(wid * npw + i * CHUNK, CHUNK)])


def kernel(seq, table):
    b, s = seq.shape
    n = b * s
    idx = seq.reshape(n // SUB, SUB)
    mesh = plsc.VectorSubcoreMesh(core_axis_name="c", subcore_axis_name="s")
    run = pl.kernel(
        _gather_body,
        out_type=jax.ShapeDtypeStruct((n, EMBED), jnp.float32),
        mesh=mesh,
        scratch_types=[
            pltpu.VMEM((NSUB, SUB), jnp.int32),
            pltpu.VMEM((CHUNK, EMBED), jnp.float32),
            pltpu.SemaphoreType.DMA,
        ],
        compiler_params=pltpu.CompilerParams(use_tc_tiling_on_sc=False),
    )
    out = run(table, idx)
    return out.reshape(b, s, EMBED)


# speedup vs baseline: 1.0460x; 1.0460x over previous
import jax
import jax.numpy as jnp
from jax import lax
from jax.experimental import pallas as pl
from jax.experimental.pallas import tpu as pltpu
from jax.experimental.pallas import tpu_sc as plsc

EMBED = 64
NC, NS = 2, 16
NW = NC * NS
CHUNK = 512
NBUF = 2


def _gather_body(table_hbm, idx_hbm, out_hbm, idx_v, rows_v, *sems):
    idx_sems = sems[0:NBUF]
    gat_sems = sems[NBUF:2 * NBUF]
    out_sems = sems[2 * NBUF:3 * NBUF]
    wid = lax.axis_index("s") * NC + lax.axis_index("c")
    n = idx_hbm.shape[0]
    npw = n // NW
    nchunks = npw // CHUNK
    base0 = wid * npw

    def idx_src(ci):
        return idx_hbm.at[pl.ds(base0 + ci * CHUNK, CHUNK)]

    def out_dst(ci):
        return out_hbm.at[pl.ds(base0 + ci * CHUNK, CHUNK)]

    for b in range(NBUF):
        pltpu.async_copy(idx_src(b), idx_v.at[b], idx_sems[b])

    @pl.loop(0, nchunks, step=NBUF)
    def _step(i):
        for b in range(NBUF):
            ci = i + b

            @pl.when(ci >= NBUF)
            def _():
                pltpu.make_async_copy(rows_v.at[b], out_dst(ci), out_sems[b]).wait()

            pltpu.make_async_copy(idx_src(ci), idx_v.at[b], idx_sems[b]).wait()

            pltpu.async_copy(
                table_hbm.at[idx_v.at[b]], rows_v.at[b], gat_sems[b]
            ).wait()

            @pl.when(ci + NBUF < nchunks)
            def _():
                pltpu.async_copy(idx_src(ci + NBUF), idx_v.at[b], idx_sems[b])

            pltpu.async_copy(rows_v.at[b], out_dst(ci), out_sems[b])

    for b in range(NBUF):
        pltpu.make_async_copy(
            rows_v.at[b], out_dst(nchunks - NBUF + b), out_sems[b]
        ).wait()


def kernel(seq, table):
    b, s = seq.shape
    n = b * s
    idx = seq.reshape(n)
    mesh = plsc.VectorSubcoreMesh(core_axis_name="c", subcore_axis_name="s")
    run = pl.kernel(
        _gather_body,
        out_type=jax.ShapeDtypeStruct((n, EMBED), jnp.float32),
        mesh=mesh,
        scratch_types=[
            pltpu.VMEM((NBUF, CHUNK), jnp.int32),
            pltpu.VMEM((NBUF, CHUNK, EMBED), jnp.float32),
        ] + [pltpu.SemaphoreType.DMA] * (3 * NBUF),
        compiler_params=pltpu.CompilerParams(use_tc_tiling_on_sc=False),
    )
    out = run(table, idx)
    return out.reshape(b, s, EMBED)
